# baseline (speedup 1.0000x reference)
import functools

import jax
import jax.numpy as jnp
from jax import lax
from jax.experimental import pallas as pl
from jax.experimental.pallas import tpu as pltpu
from jax.experimental.pallas import tpu_sc as plsc

TOPK = 2
BT = 256
NC, NS = 2, 16
NW = NC * NS


def _router_body(x_ref, gate_ref, logits_ref, s0_ref, s1_ref,
                 w0_ref, w1_ref, be_ref, bm_ref, nb_ref, NB):
    T = x_ref.shape[0]
    E = gate_ref.shape[0]
    logits = lax.dot_general(x_ref[...], gate_ref[...],
                             (((1,), (1,)), ((), ())),
                             preferred_element_type=jnp.float32)
    logits_ref[...] = logits
    p = jax.nn.softmax(logits, axis=1)
    iota = lax.broadcasted_iota(jnp.int32, p.shape, 1)
    m1 = jnp.max(p, axis=1, keepdims=True)
    e0 = jnp.min(jnp.where(p >= m1, iota, E), axis=1, keepdims=True)
    p2 = jnp.where(p >= m1, -jnp.inf, p)
    m2 = jnp.max(p2, axis=1, keepdims=True)
    e1 = jnp.min(jnp.where(p2 >= m2, iota, E), axis=1, keepdims=True)
    denom = m1 + m2
    w0_ref[...] = jnp.broadcast_to(m1 / denom, w0_ref.shape)
    w1_ref[...] = jnp.broadcast_to(m2 / denom, w1_ref.shape)

    onehot0 = (iota == e0).astype(jnp.float32)
    onehot1 = (iota == e1).astype(jnp.float32)
    oh = onehot0 + onehot1
    row_i = lax.broadcasted_iota(jnp.int32, (T, T), 0)
    col_i = lax.broadcasted_iota(jnp.int32, (T, T), 1)
    lstrict = (row_i > col_i).astype(jnp.float32)
    c_excl = lax.dot_general(lstrict, oh, (((1,), (0,)), ((), ())),
                             preferred_element_type=jnp.float32)

    cnt = jnp.sum(oh, axis=0, keepdims=True)
    blocks = jnp.floor((cnt + (BT - 1)) / BT)
    lt8_r = lax.broadcasted_iota(jnp.int32, (E, E), 0)
    lt8_c = lax.broadcasted_iota(jnp.int32, (E, E), 1)
    lt8 = (lt8_r <= lt8_c).astype(jnp.float32)
    cblocks = lax.dot_general(blocks, lt8, (((1,), (0,)), ((), ())),
                              preferred_element_type=jnp.float32)
    pad_off = (cblocks - blocks) * BT

    rank0 = jnp.sum(onehot0 * c_excl, axis=1, keepdims=True)
    rank1 = jnp.sum(onehot1 * c_excl, axis=1, keepdims=True)
    off0 = jnp.sum(onehot0 * pad_off, axis=1, keepdims=True)
    off1 = jnp.sum(onehot1 * pad_off, axis=1, keepdims=True)
    s0_ref[...] = (rank0 + off0).astype(jnp.int32)
    s1_ref[...] = (rank1 + off1).astype(jnp.int32)

    nb = cblocks[0:1, E - 1:E].astype(jnp.int32)
    bidx = lax.broadcasted_iota(jnp.int32, (NB, 1), 0)
    cb_b = jnp.broadcast_to(cblocks, (NB, E))
    be_raw = jnp.sum((cb_b <= bidx.astype(jnp.float32)).astype(jnp.int32),
                     axis=1, keepdims=True)
    be_last = jnp.sum(
        (cblocks <= (nb - 1).astype(jnp.float32)).astype(jnp.int32),
        axis=1, keepdims=True)
    active = bidx < nb
    be_ref[...] = jnp.where(active, jnp.minimum(be_raw, E - 1),
                            jnp.broadcast_to(be_last, (NB, 1)))
    bm_ref[...] = jnp.where(active, bidx, jnp.broadcast_to(nb - 1, (NB, 1)))
    nb_ref[...] = nb


def _router(x, gate_w, NB):
    T, H = x.shape
    E = gate_w.shape[0]
    return pl.pallas_call(
        functools.partial(_router_body, NB=NB),
        out_shape=[
            jax.ShapeDtypeStruct((T, E), jnp.float32),
            jax.ShapeDtypeStruct((T, 1), jnp.int32),
            jax.ShapeDtypeStruct((T, 1), jnp.int32),
            jax.ShapeDtypeStruct((T, 16), jnp.float32),
            jax.ShapeDtypeStruct((T, 16), jnp.float32),
            jax.ShapeDtypeStruct((NB, 1), jnp.int32),
            jax.ShapeDtypeStruct((NB, 1), jnp.int32),
            jax.ShapeDtypeStruct((1, 1), jnp.int32),
        ],
    )(x, gate_w)


def _sc_dispatch(x, s0, s1, P_max):
    T, H = x.shape
    rows_per_w = T // NW
    mesh = plsc.VectorSubcoreMesh(core_axis_name="c", subcore_axis_name="s")

    @functools.partial(
        pl.kernel, mesh=mesh,
        out_type=jax.ShapeDtypeStruct((P_max, H), jnp.float32),
        scratch_types=[
            pltpu.VMEM((rows_per_w,), jnp.int32),
            pltpu.VMEM((rows_per_w,), jnp.int32),
            pltpu.VMEM((rows_per_w, H), jnp.float32),
            pltpu.SemaphoreType.DMA,
            pltpu.SemaphoreType.DMA,
        ],
    )
    def k(x_hbm, s0_hbm, s1_hbm, xs_hbm, i0_v, i1_v, xbuf, sa, sb):
        wid = lax.axis_index("s") * NC + lax.axis_index("c")
        base = wid * rows_per_w
        pltpu.sync_copy(s0_hbm.at[pl.ds(base, rows_per_w)], i0_v)
        pltpu.sync_copy(s1_hbm.at[pl.ds(base, rows_per_w)], i1_v)
        pltpu.sync_copy(x_hbm.at[pl.ds(base, rows_per_w)], xbuf)
        ca = pltpu.async_copy(xbuf, xs_hbm.at[i0_v], sa)
        cb = pltpu.async_copy(xbuf, xs_hbm.at[i1_v], sb)
        ca.wait()
        cb.wait()

    return k(x, s0, s1)


def _ffn_body(be_ref, bm_ref, nb_ref, xs_ref, w1_ref, w3_ref, w2_ref, ys_ref):
    b = pl.program_id(0)

    @pl.when(b < nb_ref[0])
    def _():
        xs = xs_ref[...].astype(jnp.bfloat16)
        h = jax.nn.silu(
            jnp.dot(xs, w1_ref[0], preferred_element_type=jnp.float32)
        ) * jnp.dot(xs, w3_ref[0], preferred_element_type=jnp.float32)
        hb = h.astype(jnp.bfloat16)
        ys_ref[...] = jnp.dot(hb, w2_ref[0],
                              preferred_element_type=jnp.float32)


def _grouped_ffn(xs, w1, w3, w2, block_e, block_m, nb):
    P_max, H = xs.shape
    E, _, FFN = w1.shape
    NB = P_max // BT
    grid_spec = pltpu.PrefetchScalarGridSpec(
        num_scalar_prefetch=3,
        grid=(NB,),
        in_specs=[
            pl.BlockSpec((BT, H), lambda b, be, bm, nb: (bm[b], 0)),
            pl.BlockSpec((1, H, FFN), lambda b, be, bm, nb: (be[b], 0, 0)),
            pl.BlockSpec((1, H, FFN), lambda b, be, bm, nb: (be[b], 0, 0)),
            pl.BlockSpec((1, FFN, H), lambda b, be, bm, nb: (be[b], 0, 0)),
        ],
        out_specs=pl.BlockSpec((BT, H), lambda b, be, bm, nb: (bm[b], 0)),
    )
    return pl.pallas_call(
        _ffn_body,
        grid_spec=grid_spec,
        out_shape=jax.ShapeDtypeStruct((P_max, H), jnp.float32),
        compiler_params=pltpu.CompilerParams(
            dimension_semantics=("arbitrary",)),
    )(block_e, block_m, nb, xs, w1, w3, w2)


def _sc_combine(ys, p0, p1, w0m, w1m, T):
    H = ys.shape[1]
    rows_per_w = T // NW
    CW = 32
    nch = rows_per_w // CW
    mesh = plsc.VectorSubcoreMesh(core_axis_name="c", subcore_axis_name="s")

    @functools.partial(
        pl.kernel, mesh=mesh,
        out_type=jax.ShapeDtypeStruct((T, H), jnp.float32),
        scratch_types=[
            pltpu.VMEM((rows_per_w,), jnp.int32),
            pltpu.VMEM((rows_per_w,), jnp.int32),
            pltpu.VMEM((rows_per_w, 16), jnp.float32),
            pltpu.VMEM((rows_per_w, 16), jnp.float32),
            pltpu.VMEM((CW, H), jnp.float32),
            pltpu.VMEM((CW, H), jnp.float32),
            pltpu.SemaphoreType.DMA,
            pltpu.SemaphoreType.DMA,
        ],
    )
    def k(ys_hbm, p0_hbm, p1_hbm, w0_hbm, w1_hbm, out_hbm,
          i0_v, i1_v, w0_v, w1_v, bufa, bufb, sa, sb):
        wid = lax.axis_index("s") * NC + lax.axis_index("c")
        base = wid * rows_per_w
        pltpu.sync_copy(p0_hbm.at[pl.ds(base, rows_per_w)], i0_v)
        pltpu.sync_copy(p1_hbm.at[pl.ds(base, rows_per_w)], i1_v)
        pltpu.sync_copy(w0_hbm.at[pl.ds(base, rows_per_w)], w0_v)
        pltpu.sync_copy(w1_hbm.at[pl.ds(base, rows_per_w)], w1_v)
        for ch in range(nch):
            ca = pltpu.async_copy(
                ys_hbm.at[i0_v.at[pl.ds(ch * CW, CW)]], bufa, sa)
            cb = pltpu.async_copy(
                ys_hbm.at[i1_v.at[pl.ds(ch * CW, CW)]], bufb, sb)
            ca.wait()
            cb.wait()

            def row_fma(r, _):
                rr = ch * CW + r
                wv0 = w0_v[rr, :]
                wv1 = w1_v[rr, :]
                for c in range(H // 16):
                    sl = pl.ds(c * 16, 16)
                    bufa[r, sl] = bufa[r, sl] * wv0 + bufb[r, sl] * wv1
                return 0

            lax.fori_loop(0, CW, row_fma, 0)
            pltpu.sync_copy(bufa, out_hbm.at[pl.ds(base + ch * CW, CW)])

    return k(ys, p0, p1, w0m, w1m)


def kernel(hidden_states, gate_w, w1, w2, w3):
    batch, seq, hidden = hidden_states.shape
    T = batch * seq
    E, _, FFN = w1.shape
    P = T * TOPK
    NB = P // BT + E
    P_max = NB * BT
    x = hidden_states.reshape(T, hidden)

    logits, s0, s1, w0m, w1m, block_e, block_m, nb = _router(x, gate_w, NB)
    s0 = s0.reshape(T)
    s1 = s1.reshape(T)

    xs = _sc_dispatch(x, s0, s1, P_max)
    ys = _grouped_ffn(xs, w1.astype(jnp.bfloat16), w3.astype(jnp.bfloat16),
                      w2.astype(jnp.bfloat16),
                      block_e.reshape(NB), block_m.reshape(NB), nb.reshape(1))
    out = _sc_combine(ys, s0, s1, w0m, w1m, T)

    return out.reshape(batch, seq, hidden), logits


# speedup vs baseline: 1.3627x; 1.3627x over previous
import functools

import jax
import jax.numpy as jnp
from jax import lax
from jax.experimental import pallas as pl
from jax.experimental.pallas import tpu as pltpu
from jax.experimental.pallas import tpu_sc as plsc

TOPK = 2
BT = 256
NC, NS = 2, 16
NW = NC * NS


def _router_body(x_ref, gate_ref, logits_ref, s0_ref, s1_ref,
                 w0_ref, w1_ref, be_ref, bm_ref, nb_ref, NB):
    T = x_ref.shape[0]
    E = gate_ref.shape[0]
    logits = lax.dot_general(x_ref[...], gate_ref[...],
                             (((1,), (1,)), ((), ())),
                             preferred_element_type=jnp.float32)
    logits_ref[...] = logits
    p = jax.nn.softmax(logits, axis=1)
    iota = lax.broadcasted_iota(jnp.int32, p.shape, 1)
    m1 = jnp.max(p, axis=1, keepdims=True)
    e0 = jnp.min(jnp.where(p >= m1, iota, E), axis=1, keepdims=True)
    p2 = jnp.where(p >= m1, -jnp.inf, p)
    m2 = jnp.max(p2, axis=1, keepdims=True)
    e1 = jnp.min(jnp.where(p2 >= m2, iota, E), axis=1, keepdims=True)
    denom = m1 + m2
    w0_ref[...] = jnp.broadcast_to(m1 / denom, w0_ref.shape)
    w1_ref[...] = jnp.broadcast_to(m2 / denom, w1_ref.shape)

    onehot0 = (iota == e0).astype(jnp.float32)
    onehot1 = (iota == e1).astype(jnp.float32)
    oh = onehot0 + onehot1
    row_i = lax.broadcasted_iota(jnp.int32, (T, T), 0)
    col_i = lax.broadcasted_iota(jnp.int32, (T, T), 1)
    lstrict = (row_i > col_i).astype(jnp.float32)
    c_excl = lax.dot_general(lstrict, oh, (((1,), (0,)), ((), ())),
                             preferred_element_type=jnp.float32)

    cnt = jnp.sum(oh, axis=0, keepdims=True)
    blocks = jnp.floor((cnt + (BT - 1)) / BT)
    lt8_r = lax.broadcasted_iota(jnp.int32, (E, E), 0)
    lt8_c = lax.broadcasted_iota(jnp.int32, (E, E), 1)
    lt8 = (lt8_r <= lt8_c).astype(jnp.float32)
    cblocks = lax.dot_general(blocks, lt8, (((1,), (0,)), ((), ())),
                              preferred_element_type=jnp.float32)
    pad_off = (cblocks - blocks) * BT

    rank0 = jnp.sum(onehot0 * c_excl, axis=1, keepdims=True)
    rank1 = jnp.sum(onehot1 * c_excl, axis=1, keepdims=True)
    off0 = jnp.sum(onehot0 * pad_off, axis=1, keepdims=True)
    off1 = jnp.sum(onehot1 * pad_off, axis=1, keepdims=True)
    s0_ref[...] = (rank0 + off0).astype(jnp.int32)
    s1_ref[...] = (rank1 + off1).astype(jnp.int32)

    nb = cblocks[0:1, E - 1:E].astype(jnp.int32)
    bidx = lax.broadcasted_iota(jnp.int32, (NB, 1), 0)
    cb_b = jnp.broadcast_to(cblocks, (NB, E))
    be_raw = jnp.sum((cb_b <= bidx.astype(jnp.float32)).astype(jnp.int32),
                     axis=1, keepdims=True)
    be_last = jnp.sum(
        (cblocks <= (nb - 1).astype(jnp.float32)).astype(jnp.int32),
        axis=1, keepdims=True)
    active = bidx < nb
    be_ref[...] = jnp.where(active, jnp.minimum(be_raw, E - 1),
                            jnp.broadcast_to(be_last, (NB, 1)))
    bm_ref[...] = jnp.where(active, bidx, jnp.broadcast_to(nb - 1, (NB, 1)))
    nb_ref[...] = nb


def _router(x, gate_w, NB):
    T, H = x.shape
    E = gate_w.shape[0]
    return pl.pallas_call(
        functools.partial(_router_body, NB=NB),
        out_shape=[
            jax.ShapeDtypeStruct((T, E), jnp.float32),
            jax.ShapeDtypeStruct((T, 1), jnp.int32),
            jax.ShapeDtypeStruct((T, 1), jnp.int32),
            jax.ShapeDtypeStruct((T, 16), jnp.float32),
            jax.ShapeDtypeStruct((T, 16), jnp.float32),
            jax.ShapeDtypeStruct((NB, 1), jnp.int32),
            jax.ShapeDtypeStruct((NB, 1), jnp.int32),
            jax.ShapeDtypeStruct((1, 1), jnp.int32),
        ],
    )(x, gate_w)


def _sc_dispatch(x, s0, s1, P_max):
    T, H = x.shape
    rows_per_w = T // NW
    mesh = plsc.VectorSubcoreMesh(core_axis_name="c", subcore_axis_name="s")

    @functools.partial(
        pl.kernel, mesh=mesh,
        out_type=jax.ShapeDtypeStruct((P_max, H), jnp.float32),
        scratch_types=[
            pltpu.VMEM((rows_per_w,), jnp.int32),
            pltpu.VMEM((rows_per_w,), jnp.int32),
            pltpu.VMEM((rows_per_w, H), jnp.float32),
            pltpu.SemaphoreType.DMA,
            pltpu.SemaphoreType.DMA,
        ],
    )
    def k(x_hbm, s0_hbm, s1_hbm, xs_hbm, i0_v, i1_v, xbuf, sa, sb):
        wid = lax.axis_index("s") * NC + lax.axis_index("c")
        base = wid * rows_per_w
        pltpu.sync_copy(s0_hbm.at[pl.ds(base, rows_per_w)], i0_v)
        pltpu.sync_copy(s1_hbm.at[pl.ds(base, rows_per_w)], i1_v)
        pltpu.sync_copy(x_hbm.at[pl.ds(base, rows_per_w)], xbuf)
        ca = pltpu.async_copy(xbuf, xs_hbm.at[i0_v], sa)
        cb = pltpu.async_copy(xbuf, xs_hbm.at[i1_v], sb)
        ca.wait()
        cb.wait()

    return k(x, s0, s1)


def _ffn_body(be_ref, bm_ref, nb_ref, xs_ref, w1_ref, w3_ref, w2_ref, ys_ref):
    b = pl.program_id(0)

    @pl.when(b < nb_ref[0])
    def _():
        xs = xs_ref[...]
        h = jax.nn.silu(
            jnp.dot(xs, w1_ref[0], preferred_element_type=jnp.float32)
        ) * jnp.dot(xs, w3_ref[0], preferred_element_type=jnp.float32)
        ys_ref[...] = jnp.dot(h, w2_ref[0], preferred_element_type=jnp.float32)


def _grouped_ffn(xs, w1, w3, w2, block_e, block_m, nb):
    P_max, H = xs.shape
    E, _, FFN = w1.shape
    NB = P_max // BT
    grid_spec = pltpu.PrefetchScalarGridSpec(
        num_scalar_prefetch=3,
        grid=(NB,),
        in_specs=[
            pl.BlockSpec((BT, H), lambda b, be, bm, nb: (bm[b], 0)),
            pl.BlockSpec((1, H, FFN), lambda b, be, bm, nb: (be[b], 0, 0)),
            pl.BlockSpec((1, H, FFN), lambda b, be, bm, nb: (be[b], 0, 0)),
            pl.BlockSpec((1, FFN, H), lambda b, be, bm, nb: (be[b], 0, 0)),
        ],
        out_specs=pl.BlockSpec((BT, H), lambda b, be, bm, nb: (bm[b], 0)),
    )
    return pl.pallas_call(
        _ffn_body,
        grid_spec=grid_spec,
        out_shape=jax.ShapeDtypeStruct((P_max, H), jnp.float32),
        compiler_params=pltpu.CompilerParams(
            dimension_semantics=("arbitrary",)),
    )(block_e, block_m, nb, xs, w1, w3, w2)


def _sc_combine(ys, p0, p1, w0m, w1m, T):
    H = ys.shape[1]
    rows_per_w = T // NW
    CW = 16
    nch = rows_per_w // CW
    mesh = plsc.VectorSubcoreMesh(core_axis_name="c", subcore_axis_name="s")

    @functools.partial(
        pl.kernel, mesh=mesh,
        out_type=jax.ShapeDtypeStruct((T, H), jnp.float32),
        scratch_types=[
            pltpu.VMEM((rows_per_w,), jnp.int32),
            pltpu.VMEM((rows_per_w,), jnp.int32),
            pltpu.VMEM((rows_per_w, 16), jnp.float32),
            pltpu.VMEM((rows_per_w, 16), jnp.float32),
        ] + [pltpu.VMEM((CW, H), jnp.float32) for _ in range(4)]
          + [pltpu.SemaphoreType.DMA for _ in range(5)],
    )
    def k(ys_hbm, p0_hbm, p1_hbm, w0_hbm, w1_hbm, out_hbm,
          i0_v, i1_v, w0_v, w1_v, a0, b0, a1, b1, sa0, sb0, sa1, sb1, so):
        bufa = (a0, a1)
        bufb = (b0, b1)
        sa = (sa0, sa1)
        sb = (sb0, sb1)
        wid = lax.axis_index("s") * NC + lax.axis_index("c")
        base = wid * rows_per_w
        pltpu.sync_copy(p0_hbm.at[pl.ds(base, rows_per_w)], i0_v)
        pltpu.sync_copy(p1_hbm.at[pl.ds(base, rows_per_w)], i1_v)
        pltpu.sync_copy(w0_hbm.at[pl.ds(base, rows_per_w)], w0_v)
        pltpu.sync_copy(w1_hbm.at[pl.ds(base, rows_per_w)], w1_v)

        def gather_pair(ch, bi):
            ga = pltpu.async_copy(
                ys_hbm.at[i0_v.at[pl.ds(ch * CW, CW)]], bufa[bi], sa[bi])
            gb = pltpu.async_copy(
                ys_hbm.at[i1_v.at[pl.ds(ch * CW, CW)]], bufb[bi], sb[bi])
            return ga, gb

        pend = gather_pair(0, 0)
        out_pend = None
        for ch in range(nch):
            bi = ch % 2
            pend[0].wait()
            pend[1].wait()
            if out_pend is not None:
                out_pend.wait()
                out_pend = None
            if ch + 1 < nch:
                pend = gather_pair(ch + 1, 1 - bi)

            def row_fma(r, _, ch=ch, bi=bi):
                rr = ch * CW + r
                wv0 = w0_v[rr, :]
                wv1 = w1_v[rr, :]
                for c in range(H // 16):
                    sl = pl.ds(c * 16, 16)
                    bufa[bi][r, sl] = (bufa[bi][r, sl] * wv0
                                       + bufb[bi][r, sl] * wv1)
                return 0

            lax.fori_loop(0, CW, row_fma, 0)
            out_pend = pltpu.async_copy(
                bufa[bi], out_hbm.at[pl.ds(base + ch * CW, CW)], so)
        out_pend.wait()

    return k(ys, p0, p1, w0m, w1m)


def kernel(hidden_states, gate_w, w1, w2, w3):
    batch, seq, hidden = hidden_states.shape
    T = batch * seq
    E, _, FFN = w1.shape
    P = T * TOPK
    NB = P // BT + E
    P_max = NB * BT
    x = hidden_states.reshape(T, hidden)

    logits, s0, s1, w0m, w1m, block_e, block_m, nb = _router(x, gate_w, NB)
    s0 = s0.reshape(T)
    s1 = s1.reshape(T)

    xs = _sc_dispatch(x, s0, s1, P_max)
    ys = _grouped_ffn(xs, w1, w3, w2,
                      block_e.reshape(NB), block_m.reshape(NB), nb.reshape(1))
    out = _sc_combine(ys, s0, s1, w0m, w1m, T)

    return out.reshape(batch, seq, hidden), logits
